# Initial kernel scaffold; baseline (speedup 1.0000x reference)
import jax
import jax.numpy as jnp
from jax.experimental import pallas as pl


def kernel(fushed_features, input_data, adj_mx_topk_index, W, a):
    raise NotImplementedError("write your pallas kernel here")


# speedup vs baseline: 3.4117x; 3.4117x over previous
import functools

import jax
import jax.numpy as jnp
from jax import lax
from jax.experimental import pallas as pl
from jax.experimental.pallas import tpu as pltpu
from jax.experimental.pallas import tpu_sc as plsc

_NC, _NS = 2, 16
_NW = _NC * _NS
_LN = 16


def _sc_attention(xT, gidx, w1b, w2b, *, rpad, rpw, kp, K, L):
    ow = K * L
    groups = rpw // _LN
    kv1 = K - _LN

    mesh = plsc.VectorSubcoreMesh(
        core_axis_name="c", subcore_axis_name="s",
        num_cores=_NC, num_subcores=_NS)

    @functools.partial(
        pl.kernel,
        out_type=(jax.ShapeDtypeStruct((rpad * ow,), jnp.float32),
                  jax.ShapeDtypeStruct((_NW * _LN,), jnp.float32)),
        mesh=mesh,
        compiler_params=pltpu.CompilerParams(needs_layout_passes=False),
        scratch_types=[
            pltpu.VMEM((L * rpad,), jnp.float32),
            pltpu.VMEM((rpw * kp,), jnp.int32),
            pltpu.VMEM((rpad,), jnp.float32),
            pltpu.VMEM((-(-rpw // 128) * 128,), jnp.float32),
            pltpu.VMEM((L * _LN,), jnp.float32),
            pltpu.VMEM((L * _LN,), jnp.float32),
            pltpu.VMEM((_LN * ow,), jnp.float32),
            pltpu.VMEM((_LN * ow,), jnp.float32),
            pltpu.VMEM((_LN,), jnp.float32),
            pltpu.SemaphoreType.DMA,
            pltpu.SemaphoreType.DMA,
        ],
    )
    def run(xT_h, gidx_h, w1_h, w2_h, out_h, part_h,
            xT_v, gidx_v, f_v, e_v, w1_v, w2_v, ob0, ob1, acc_v,
            sem0, sem1):
        wid = lax.axis_index("s") * _NC + lax.axis_index("c")
        base = wid * rpw

        pltpu.sync_copy(xT_h, xT_v)
        pltpu.sync_copy(gidx_h.at[pl.ds(base * kp, rpw * kp)], gidx_v)
        pltpu.sync_copy(w1_h, w1_v)
        pltpu.sync_copy(w2_h, w2_v)

        w1vec = [w1_v[pl.ds(l * _LN, _LN)] for l in range(L)]
        w2vec = [w2_v[pl.ds(l * _LN, _LN)] for l in range(L)]

        @pl.loop(0, rpad // _LN)
        def _f(c):
            o = c * _LN
            fv = xT_v[pl.ds(o, _LN)] * w2vec[0]
            for l in range(1, L):
                fv = fv + xT_v[pl.ds(l * rpad + o, _LN)] * w2vec[l]
            f_v[pl.ds(o, _LN)] = fv

        for ce in range(groups):
            o = base + ce * _LN
            ev = xT_v[pl.ds(o, _LN)] * w1vec[0]
            for l in range(1, L):
                ev = ev + xT_v[pl.ds(l * rpad + o, _LN)] * w1vec[l]
            e_v[pl.ds(ce * _LN, _LN)] = ev

        acc_v[...] = jnp.zeros((_LN,), jnp.float32)
        lane = lax.iota(jnp.int32, _LN)
        maskv = lane < kv1
        ki0 = lane * L
        ki1 = (lane + _LN) * L

        obufs = (ob0, ob1)
        sems = (sem0, sem1)
        handles = [None, None]

        for g in range(groups):
            ob = obufs[g % 2]
            if handles[g % 2] is not None:
                handles[g % 2].wait()

            @pl.loop(0, _LN)
            def _row(t):
                r = g * _LN + t
                e_s = plsc.load_gather(e_v, [jnp.broadcast_to(r, (_LN,))])
                j0 = gidx_v[pl.ds(r * kp, _LN)]
                j1 = gidx_v[pl.ds(r * kp + _LN, _LN)]
                s0 = e_s + plsc.load_gather(f_v, [j0])
                s1 = e_s + plsc.load_gather(f_v, [j1])
                s0 = jnp.maximum(s0, 0.5 * s0)
                s1 = jnp.maximum(s1, 0.5 * s1)
                mx = jnp.max(jnp.maximum(s0, jnp.where(maskv, s1, -3e38)))
                p0 = jnp.exp(s0 - mx)
                p1 = jnp.where(maskv, jnp.exp(s1 - mx), 0.0)
                sm = jnp.sum(p0 + p1)
                a0 = p0 / sm
                a1 = p1 / sm
                o0 = t * ow + ki0
                o1 = t * ow + ki1
                rs = jnp.zeros((_LN,), jnp.float32)
                for l in range(L):
                    x0 = plsc.load_gather(xT_v, [j0 + (l * rpad)])
                    x1 = plsc.load_gather(xT_v, [j1 + (l * rpad)])
                    v0 = a0 * x0
                    v1 = a1 * x1
                    plsc.store_scatter(ob, [o0 + l], v0)
                    plsc.store_scatter(ob, [o1 + l], v1, mask=maskv)
                    rs = rs + v0 + v1
                plsc.addupdate(acc_v.at[pl.ds(0, _LN)], rs)

            handles[g % 2] = pltpu.async_copy(
                ob, out_h.at[pl.ds((base + g * _LN) * ow, _LN * ow)],
                sems[g % 2])

        for h in handles:
            if h is not None:
                h.wait()
        pltpu.sync_copy(acc_v, part_h.at[pl.ds(wid * _LN, _LN)])

    return run(xT, gidx, w1b, w2b)


def kernel(fushed_features, input_data, adj_mx_topk_index, W, a):
    del fushed_features
    B, N, L = input_data.shape
    K = adj_mx_topk_index.shape[2]
    S = W.shape[1]
    R = B * N
    kp = 2 * _LN
    rpw = -(-R // (_NW * _LN)) * _LN
    rpad = _NW * rpw

    w1 = W @ a[:S, 0]
    w2 = W @ a[S:, 0]
    w1b = jnp.tile(w1[:, None], (1, _LN)).reshape(-1)
    w2b = jnp.tile(w2[:, None], (1, _LN)).reshape(-1)

    x2 = jnp.pad(input_data.reshape(R, L), ((0, rpad - R), (0, 0)))
    xT = x2.T.reshape(-1)

    gidx = (adj_mx_topk_index.astype(jnp.int32)
            + (jnp.arange(B, dtype=jnp.int32) * N)[:, None, None])
    gidx = jnp.pad(gidx.reshape(R, K), ((0, rpad - R), (0, kp - K)),
                   constant_values=R)

    out_flat, part = _sc_attention(
        xT, gidx.reshape(-1), w1b, w2b,
        rpad=rpad, rpw=rpw, kp=kp, K=K, L=L)

    out = out_flat.reshape(rpad, K, L)[:R].reshape(B, N, K, L)
    m = jnp.sum(part) / (B * N * K * L)
    return (out, m, m, m)


# speedup vs baseline: 3.4936x; 1.0240x over previous
import functools

import jax
import jax.numpy as jnp
from jax import lax
from jax.experimental import pallas as pl
from jax.experimental.pallas import tpu as pltpu
from jax.experimental.pallas import tpu_sc as plsc

_NC, _NS = 2, 16
_NW = _NC * _NS
_LN = 16


def _sc_attention(xT, gidx, w1b, w2b, *, R, rpad, rpw, kp, K, L):
    ow = K * L
    groups = rpw // _LN
    kv1 = K - _LN

    mesh = plsc.VectorSubcoreMesh(
        core_axis_name="c", subcore_axis_name="s",
        num_cores=_NC, num_subcores=_NS)

    @functools.partial(
        pl.kernel,
        out_type=(jax.ShapeDtypeStruct((R * ow,), jnp.float32),
                  jax.ShapeDtypeStruct((_NW * _LN,), jnp.float32)),
        mesh=mesh,
        compiler_params=pltpu.CompilerParams(needs_layout_passes=False),
        scratch_types=[
            pltpu.VMEM((L * rpad,), jnp.float32),
            pltpu.VMEM((rpw * kp,), jnp.int32),
            pltpu.VMEM((rpad,), jnp.float32),
            pltpu.VMEM((-(-rpw // 128) * 128,), jnp.float32),
            pltpu.VMEM((L * _LN,), jnp.float32),
            pltpu.VMEM((L * _LN,), jnp.float32),
            pltpu.VMEM((_LN * ow,), jnp.float32),
            pltpu.VMEM((_LN * ow,), jnp.float32),
            pltpu.VMEM((_LN,), jnp.float32),
            pltpu.SemaphoreType.DMA,
            pltpu.SemaphoreType.DMA,
        ],
    )
    def run(xT_h, gidx_h, w1_h, w2_h, out_h, part_h,
            xT_v, gidx_v, f_v, e_v, w1_v, w2_v, ob0, ob1, acc_v,
            sem0, sem1):
        wid = lax.axis_index("s") * _NC + lax.axis_index("c")
        base = wid * rpw

        pltpu.sync_copy(xT_h, xT_v)
        pltpu.sync_copy(gidx_h.at[pl.ds(base * kp, rpw * kp)], gidx_v)
        pltpu.sync_copy(w1_h, w1_v)
        pltpu.sync_copy(w2_h, w2_v)

        w1vec = [w1_v[pl.ds(l * _LN, _LN)] for l in range(L)]
        w2vec = [w2_v[pl.ds(l * _LN, _LN)] for l in range(L)]

        @pl.loop(0, rpad // _LN)
        def _f(c):
            o = c * _LN
            fv = xT_v[pl.ds(o, _LN)] * w2vec[0]
            for l in range(1, L):
                fv = fv + xT_v[pl.ds(l * rpad + o, _LN)] * w2vec[l]
            f_v[pl.ds(o, _LN)] = fv

        for ce in range(groups):
            o = base + ce * _LN
            ev = xT_v[pl.ds(o, _LN)] * w1vec[0]
            for l in range(1, L):
                ev = ev + xT_v[pl.ds(l * rpad + o, _LN)] * w1vec[l]
            e_v[pl.ds(ce * _LN, _LN)] = ev

        acc_v[...] = jnp.zeros((_LN,), jnp.float32)
        lane = lax.iota(jnp.int32, _LN)
        maskv = lane < kv1
        ki0 = lane * L
        ki1 = (lane + _LN) * L

        obufs = (ob0, ob1)
        sems = (sem0, sem1)
        handles = [None, None]

        for g in range(groups):
            ob = obufs[g % 2]
            row0 = (base + g * _LN) * ow
            if handles[g % 2] is not None:
                prev_row0, prev_h = handles[g % 2]

                @pl.when(prev_row0 < R * ow)
                def _wait_prev():
                    prev_h.wait()

            @pl.loop(0, _LN)
            def _row(t):
                r = g * _LN + t
                e_s = plsc.load_gather(e_v, [jnp.broadcast_to(r, (_LN,))])
                j0 = gidx_v[pl.ds(r * kp, _LN)]
                j1 = gidx_v[pl.ds(r * kp + _LN, _LN)]
                s0 = e_s + plsc.load_gather(f_v, [j0])
                s1 = e_s + plsc.load_gather(f_v, [j1])
                s0 = jnp.maximum(s0, 0.5 * s0)
                s1 = jnp.maximum(s1, 0.5 * s1)
                mx = jnp.max(jnp.maximum(s0, jnp.where(maskv, s1, -3e38)))
                p0 = jnp.exp(s0 - mx)
                p1 = jnp.where(maskv, jnp.exp(s1 - mx), 0.0)
                sm = jnp.sum(p0 + p1)
                a0 = p0 / sm
                a1 = p1 / sm
                o0 = t * ow + ki0
                o1 = t * ow + ki1
                rs = jnp.zeros((_LN,), jnp.float32)
                for l in range(L):
                    x0 = plsc.load_gather(xT_v, [j0 + (l * rpad)])
                    x1 = plsc.load_gather(xT_v, [j1 + (l * rpad)])
                    v0 = a0 * x0
                    v1 = a1 * x1
                    plsc.store_scatter(ob, [o0 + l], v0)
                    plsc.store_scatter(ob, [o1 + l], v1, mask=maskv)
                    rs = rs + v0 + v1
                plsc.addupdate(acc_v.at[pl.ds(0, _LN)], rs)

            sem = sems[g % 2]

            @pl.when(row0 < R * ow)
            def _fire():
                handles[g % 2] = (
                    row0, pltpu.async_copy(
                        ob, out_h.at[pl.ds(row0, _LN * ow)], sem))

        for hr in handles:
            if hr is not None:
                last_row0, last_h = hr

                @pl.when(last_row0 < R * ow)
                def _drain():
                    last_h.wait()
        pltpu.sync_copy(acc_v, part_h.at[pl.ds(wid * _LN, _LN)])

    return run(xT, gidx, w1b, w2b)


def kernel(fushed_features, input_data, adj_mx_topk_index, W, a):
    del fushed_features
    B, N, L = input_data.shape
    K = adj_mx_topk_index.shape[2]
    S = W.shape[1]
    R = B * N
    kp = 2 * _LN
    rpw = -(-R // (_NW * _LN)) * _LN
    rpad = _NW * rpw

    w1 = jnp.matmul(W, a[:S, 0], precision="highest")
    w2 = jnp.matmul(W, a[S:, 0], precision="highest")
    w1b = jnp.tile(w1[:, None], (1, _LN)).reshape(-1)
    w2b = jnp.tile(w2[:, None], (1, _LN)).reshape(-1)

    x2 = jnp.pad(input_data.reshape(R, L), ((0, rpad - R), (0, 0)))
    xT = x2.T.reshape(-1)

    gidx = (adj_mx_topk_index.astype(jnp.int32)
            + (jnp.arange(B, dtype=jnp.int32) * N)[:, None, None])
    gidx = jnp.pad(gidx.reshape(R, K), ((0, rpad - R), (0, kp - K)),
                   constant_values=R)

    out_flat, part = _sc_attention(
        xT, gidx.reshape(-1), w1b, w2b,
        R=R, rpad=rpad, rpw=rpw, kp=kp, K=K, L=L)

    out = out_flat.reshape(B, N, K, L)
    m = jnp.sum(part) / (B * N * K * L)
    return (out, m, m, m)


# speedup vs baseline: 5.0037x; 1.4322x over previous
import functools

import jax
import jax.numpy as jnp
import numpy as np
from jax import lax
from jax.experimental import pallas as pl
from jax.experimental.pallas import tpu as pltpu
from jax.experimental.pallas import tpu_sc as plsc

_NC, _NS = 2, 16
_NW = _NC * _NS
_LN = 16


def _window_starts(N, B, rpw, rpad, win):
    base = np.arange(_NW) * rpw
    bstart = base // N
    win0 = (bstart * N) & ~15
    return np.minimum(win0, rpad - win)


def _sc_attention(xT, gidx, w1b, w2b, *, N, rpad, rpw, kp, K, L, win):
    ow = K * L
    groups = rpw // _LN
    kv1 = K - _LN

    mesh = plsc.VectorSubcoreMesh(
        core_axis_name="c", subcore_axis_name="s",
        num_cores=_NC, num_subcores=_NS)

    @functools.partial(
        pl.kernel,
        out_type=(jax.ShapeDtypeStruct((rpad * ow,), jnp.float32),
                  jax.ShapeDtypeStruct((_NW * _LN,), jnp.float32)),
        mesh=mesh,
        compiler_params=pltpu.CompilerParams(needs_layout_passes=False),
        scratch_types=[
            pltpu.VMEM((L * win,), jnp.float32),
            pltpu.VMEM((rpw * kp,), jnp.int32),
            pltpu.VMEM((win,), jnp.float32),
            pltpu.VMEM((-(-rpw // 128) * 128,), jnp.float32),
            pltpu.VMEM((L * _LN,), jnp.float32),
            pltpu.VMEM((L * _LN,), jnp.float32),
            pltpu.VMEM((ow * rpw,), jnp.float32),
            pltpu.VMEM((_LN,), jnp.float32),
        ],
    )
    def run(xT_h, gidx_h, w1_h, w2_h, out_h, part_h,
            xT_v, gidx_v, f_v, e_v, w1_v, w2_v, ob, acc_v):
        wid = lax.axis_index("s") * _NC + lax.axis_index("c")
        base = wid * rpw
        win0 = pl.multiple_of(
            jnp.minimum(((base // N) * N) & ~15, rpad - win), 16)

        for l in range(L):
            pltpu.sync_copy(xT_h.at[pl.ds(l * rpad + win0, win)],
                            xT_v.at[pl.ds(l * win, win)])
        pltpu.sync_copy(gidx_h.at[pl.ds(base * kp, rpw * kp)], gidx_v)
        pltpu.sync_copy(w1_h, w1_v)
        pltpu.sync_copy(w2_h, w2_v)

        w1vec = [w1_v[pl.ds(l * _LN, _LN)] for l in range(L)]
        w2vec = [w2_v[pl.ds(l * _LN, _LN)] for l in range(L)]

        @pl.loop(0, win // _LN)
        def _f(c):
            o = c * _LN
            fv = xT_v[pl.ds(o, _LN)] * w2vec[0]
            for l in range(1, L):
                fv = fv + xT_v[pl.ds(l * win + o, _LN)] * w2vec[l]
            f_v[pl.ds(o, _LN)] = fv

        loff = base - win0
        for ce in range(groups):
            o = loff + ce * _LN
            ev = xT_v[pl.ds(o, _LN)] * w1vec[0]
            for l in range(1, L):
                ev = ev + xT_v[pl.ds(l * win + o, _LN)] * w1vec[l]
            e_v[pl.ds(ce * _LN, _LN)] = ev

        acc_v[...] = jnp.zeros((_LN,), jnp.float32)
        lane = lax.iota(jnp.int32, _LN)
        maskv = lane < kv1
        ki0 = lane * (L * rpw)
        ki1 = (lane + _LN) * (L * rpw)

        @pl.loop(0, rpw)
        def _row(r):
            e_s = plsc.load_gather(e_v, [jnp.broadcast_to(r, (_LN,))])
            j0 = gidx_v[pl.ds(r * kp, _LN)]
            j1 = gidx_v[pl.ds(r * kp + _LN, _LN)]
            s0 = e_s + plsc.load_gather(f_v, [j0])
            s1 = e_s + plsc.load_gather(f_v, [j1])
            s0 = jnp.maximum(s0, 0.5 * s0)
            s1 = jnp.maximum(s1, 0.5 * s1)
            mx = jnp.max(jnp.maximum(s0, jnp.where(maskv, s1, -3e38)))
            p0 = jnp.exp(s0 - mx)
            p1 = jnp.where(maskv, jnp.exp(s1 - mx), 0.0)
            sm = jnp.sum(p0 + p1)
            a0 = p0 / sm
            a1 = p1 / sm
            o0 = ki0 + r
            o1 = ki1 + r
            rs = jnp.zeros((_LN,), jnp.float32)
            for l in range(L):
                x0 = plsc.load_gather(xT_v, [j0 + (l * win)])
                x1 = plsc.load_gather(xT_v, [j1 + (l * win)])
                v0 = a0 * x0
                v1 = a1 * x1
                plsc.store_scatter(ob, [o0 + (l * rpw)], v0)
                plsc.store_scatter(ob, [o1 + (l * rpw)], v1, mask=maskv)
                rs = rs + v0 + v1
            plsc.addupdate(acc_v.at[pl.ds(0, _LN)], rs)

        pltpu.sync_copy(ob, out_h.at[pl.ds(wid * (ow * rpw), ow * rpw)])
        pltpu.sync_copy(acc_v, part_h.at[pl.ds(wid * _LN, _LN)])

    return run(xT, gidx, w1b, w2b)


def kernel(fushed_features, input_data, adj_mx_topk_index, W, a):
    del fushed_features
    B, N, L = input_data.shape
    K = adj_mx_topk_index.shape[2]
    S = W.shape[1]
    R = B * N
    kp = 2 * _LN
    rpw = -(-R // (_NW * _LN)) * _LN
    rpad = _NW * rpw
    win = -(-(2 * N + 16) // _LN) * _LN

    w1 = jnp.matmul(W, a[:S, 0], precision="highest")
    w2 = jnp.matmul(W, a[S:, 0], precision="highest")
    w1b = jnp.tile(w1[:, None], (1, _LN)).reshape(-1)
    w2b = jnp.tile(w2[:, None], (1, _LN)).reshape(-1)

    x2 = jnp.pad(input_data.reshape(R, L), ((0, rpad - R), (0, 0)))
    xT = x2.T.reshape(-1)

    win0 = _window_starts(N, B, rpw, rpad, win)
    gidx = (adj_mx_topk_index.astype(jnp.int32)
            + (jnp.arange(B, dtype=jnp.int32) * N)[:, None, None])
    gidx = jnp.pad(gidx.reshape(R, K), ((0, rpad - R), (0, kp - K)),
                   constant_values=R)
    shift = jnp.asarray(np.repeat(win0, rpw).astype(np.int32))
    gidx = jnp.clip(gidx - shift[:, None], 0, win - 1)

    out_wkr, part = _sc_attention(
        xT, gidx.reshape(-1), w1b, w2b,
        N=N, rpad=rpad, rpw=rpw, kp=kp, K=K, L=L, win=win)

    out = (out_wkr.reshape(_NW, K * L, rpw)
           .transpose(1, 0, 2).reshape(K, L, rpad)[:, :, :R]
           .reshape(K, L, B, N).transpose(2, 3, 0, 1))
    m = jnp.sum(part) / (B * N * K * L)
    return (out, m, m, m)
